# baseline (speedup 1.0000x reference)
import functools

import jax
import jax.numpy as jnp
from jax import lax
from jax.experimental import pallas as pl
from jax.experimental.pallas import tpu as pltpu
from jax.experimental.pallas import tpu_sc as plsc

N = 10000
E = 320000
D = 128
H = 128
L = 4
C = 16

NP = 10240
NC = 2
NS = 16
NW = NC * NS
EW = 10240
EPAD = NW * EW
CH = 40
KCH = EW // CH
G = 32
NGRP = KCH // G
NBUF = 4
NT = NP // NS
RB = 1024
NRB = NP // RB

_f32 = jnp.float32
_i32 = jnp.int32

_sc_mesh = plsc.VectorSubcoreMesh(core_axis_name="c", subcore_axis_name="s")


def _fill(ref, rows, value):
    v = jnp.full((16,), value, dtype=_f32)
    if ref.shape == (rows,):
        offs = list(range(0, (rows // 16) * 16, 16))
        if rows % 16:
            offs.append(rows - 16)
        for o in offs:
            ref[pl.ds(o, 16)] = v
    else:
        cols = ref.shape[1]
        for i in range(rows):
            for k in range(cols // 16):
                ref[i, pl.ds(k * 16, 16)] = v


@functools.partial(
    pl.kernel,
    mesh=_sc_mesh,
    out_type=(
        jax.ShapeDtypeStruct((NC * NP,), _f32),
        jax.ShapeDtypeStruct((NC * NP,), _f32),
    ),
    scratch_types=[
        pltpu.VMEM((NGRP, 2, G, CH), _i32),
        pltpu.VMEM((CH,), _f32),
        pltpu.VMEM((NT,), _f32),
        pltpu.VMEM_SHARED((NP,), _f32),
        pltpu.VMEM_SHARED((NP,), _f32),
    ],
)
def _deg_kernel(eidx_hbm, dego_hbm, degi_hbm,
                eidx, ones_v, zrow, sh_o, sh_i):
    cid = lax.axis_index("c")
    sid = lax.axis_index("s")
    wid = sid * NC + cid
    tbase = pl.multiple_of(sid * NT, NT)
    obase = pl.multiple_of(cid * NP + sid * NT, NT)

    pltpu.sync_copy(eidx_hbm.at[wid], eidx)
    _fill(ones_v, CH, 1.0)
    _fill(zrow, NT, 0.0)
    pltpu.sync_copy(zrow, sh_o.at[pl.ds(tbase, NT)])
    pltpu.sync_copy(zrow, sh_i.at[pl.ds(tbase, NT)])
    plsc.subcore_barrier()

    def body(g, _):
        def inner(j, _2):
            pltpu.sync_copy(ones_v, sh_o.at[eidx.at[g, 0, j]], add=True)
            pltpu.sync_copy(ones_v, sh_i.at[eidx.at[g, 1, j]], add=True)
            return _2

        return lax.fori_loop(0, G, inner, _)

    lax.fori_loop(0, NGRP, body, None)
    plsc.subcore_barrier()
    pltpu.sync_copy(sh_o.at[pl.ds(tbase, NT)], dego_hbm.at[pl.ds(obase, NT)])
    pltpu.sync_copy(sh_i.at[pl.ds(tbase, NT)], degi_hbm.at[pl.ds(obase, NT)])


@functools.partial(
    pl.kernel,
    mesh=_sc_mesh,
    out_type=jax.ShapeDtypeStruct((NC, NP, H), _f32),
    scratch_types=[
        pltpu.VMEM((2, G, CH), _i32),
    ] + [pltpu.VMEM((CH, H), _f32) for _ in range(NBUF)]
    + [pltpu.VMEM_SHARED((NP, H), _f32)]
    + [pltpu.SemaphoreType.DMA for _ in range(NBUF)],
)
def _agg_kernel(hn_hbm, eidx_hbm, out_hbm,
                idx2, r0, r1, r2, r3, sh_agg,
                g0, g1, g2, g3):
    cid = lax.axis_index("c")
    sid = lax.axis_index("s")
    wid = sid * NC + cid
    tbase = pl.multiple_of(sid * NT, NT)
    rbufs = (r0, r1, r2, r3)
    gsems = (g0, g1, g2, g3)

    _fill(r0, CH, 0.0)
    for k in range(NT // CH):
        pltpu.sync_copy(r0, sh_agg.at[pl.ds(tbase + k * CH, CH)])
    plsc.subcore_barrier()

    def group(g, _):
        pltpu.sync_copy(eidx_hbm.at[wid, g], idx2)
        for x in range(NBUF - 1):
            pltpu.async_copy(hn_hbm.at[idx2.at[0, x]], rbufs[x], gsems[x])
        for c in range(G):
            b = c % NBUF
            b2 = (c - 1) % NBUF
            pltpu.make_async_copy(hn_hbm.at[idx2.at[0, c]], rbufs[b], gsems[b]).wait()
            if c + NBUF - 1 < G:
                pltpu.async_copy(hn_hbm.at[idx2.at[0, c + NBUF - 1]],
                                 rbufs[b2], gsems[b2])
            pltpu.sync_copy(rbufs[b], sh_agg.at[idx2.at[1, c]], add=True)
        return _

    lax.fori_loop(0, NGRP, group, None)
    plsc.subcore_barrier()
    for k in range(NT // CH):
        sl = pl.ds(tbase + k * CH, CH)
        pltpu.sync_copy(sh_agg.at[sl], out_hbm.at[cid, sl])


def _norm_body(dego_ref, degi_ref, ns_ref, nd_ref):
    do = dego_ref[0] + dego_ref[1]
    di = degi_ref[0] + degi_ref[1]
    ns_ref[...] = jnp.where(do > 0, lax.rsqrt(do), 0.0)
    nd_ref[...] = jnp.where(di > 0, lax.rsqrt(di), 0.0)


def _norm_call(dego, degi):
    return pl.pallas_call(
        _norm_body,
        grid=(NRB,),
        in_specs=[
            pl.BlockSpec((NC, RB, 1), lambda i: (0, i, 0)),
            pl.BlockSpec((NC, RB, 1), lambda i: (0, i, 0)),
        ],
        out_specs=[
            pl.BlockSpec((RB, 1), lambda i: (i, 0)),
            pl.BlockSpec((RB, 1), lambda i: (i, 0)),
        ],
        out_shape=[
            jax.ShapeDtypeStruct((NP, 1), _f32),
            jax.ShapeDtypeStruct((NP, 1), _f32),
        ],
    )(dego, degi)


def _embed_body(x_ref, w_ref, b_ref, ns_ref, h_ref, hn_ref):
    h = jnp.dot(x_ref[...], w_ref[...], preferred_element_type=_f32) + b_ref[...]
    h_ref[...] = h
    hn_ref[...] = h * ns_ref[...]


def _embed_call(x, w, b, ns):
    return pl.pallas_call(
        _embed_body,
        grid=(NRB,),
        in_specs=[
            pl.BlockSpec((RB, D), lambda i: (i, 0)),
            pl.BlockSpec((D, H), lambda i: (0, 0)),
            pl.BlockSpec((1, H), lambda i: (0, 0)),
            pl.BlockSpec((RB, 1), lambda i: (i, 0)),
        ],
        out_specs=[
            pl.BlockSpec((RB, H), lambda i: (i, 0)),
            pl.BlockSpec((RB, H), lambda i: (i, 0)),
        ],
        out_shape=[
            jax.ShapeDtypeStruct((NP, H), _f32),
            jax.ShapeDtypeStruct((NP, H), _f32),
        ],
    )(x, w, b, ns)


def _layer_a_body(p_ref, nd_ref, w_ref, b_ref, y_ref, s_ref):
    i = pl.program_id(0)
    aggn = (p_ref[0] + p_ref[1]) * nd_ref[...]
    y = jnp.dot(aggn, w_ref[...], preferred_element_type=_f32) + b_ref[...]
    y_ref[...] = y
    rows = i * RB + lax.broadcasted_iota(_i32, (RB, 1), 0)
    ym = jnp.where(rows < N, y, 0.0)
    sb = jnp.concatenate(
        [jnp.sum(ym, axis=0, keepdims=True),
         jnp.sum(ym * ym, axis=0, keepdims=True)], axis=0)

    @pl.when(i == 0)
    def _():
        s_ref[...] = sb

    @pl.when(i > 0)
    def _():
        s_ref[...] += sb


def _layer_a_call(p, nd, w, b):
    return pl.pallas_call(
        _layer_a_body,
        grid=(NRB,),
        in_specs=[
            pl.BlockSpec((NC, RB, H), lambda i: (0, i, 0)),
            pl.BlockSpec((RB, 1), lambda i: (i, 0)),
            pl.BlockSpec((H, H), lambda i: (0, 0)),
            pl.BlockSpec((1, H), lambda i: (0, 0)),
        ],
        out_specs=[
            pl.BlockSpec((RB, H), lambda i: (i, 0)),
            pl.BlockSpec((2, H), lambda i: (0, 0)),
        ],
        out_shape=[
            jax.ShapeDtypeStruct((NP, H), _f32),
            jax.ShapeDtypeStruct((2, H), _f32),
        ],
    )(p, nd, w, b)


def _layer_b_body(y_ref, s_ref, g_ref, be_ref, hin_ref, ns_ref, wo_ref, acc_ref,
                  h_ref, hn_ref, accout_ref):
    s = s_ref[...]
    mean = s[0:1] * (1.0 / N)
    var = s[1:2] * (1.0 / N) - mean * mean
    rstd = lax.rsqrt(var + 1e-5)
    y = (y_ref[...] - mean) * rstd * g_ref[...] + be_ref[...]
    hb = jnp.maximum(y, 0.0) + hin_ref[...]
    h_ref[...] = hb
    hn_ref[...] = hb * ns_ref[...]
    accout_ref[...] = acc_ref[...] + jnp.dot(hb, wo_ref[...], preferred_element_type=_f32)


def _layer_b_call(y, s, g, be, hin, ns, wo, acc):
    return pl.pallas_call(
        _layer_b_body,
        grid=(NRB,),
        in_specs=[
            pl.BlockSpec((RB, H), lambda i: (i, 0)),
            pl.BlockSpec((2, H), lambda i: (0, 0)),
            pl.BlockSpec((1, H), lambda i: (0, 0)),
            pl.BlockSpec((1, H), lambda i: (0, 0)),
            pl.BlockSpec((RB, H), lambda i: (i, 0)),
            pl.BlockSpec((RB, 1), lambda i: (i, 0)),
            pl.BlockSpec((H, C), lambda i: (0, 0)),
            pl.BlockSpec((RB, C), lambda i: (i, 0)),
        ],
        out_specs=[
            pl.BlockSpec((RB, H), lambda i: (i, 0)),
            pl.BlockSpec((RB, H), lambda i: (i, 0)),
            pl.BlockSpec((RB, C), lambda i: (i, 0)),
        ],
        out_shape=[
            jax.ShapeDtypeStruct((NP, H), _f32),
            jax.ShapeDtypeStruct((NP, H), _f32),
            jax.ShapeDtypeStruct((NP, C), _f32),
        ],
    )(y, s, g, be, hin, ns, wo, acc)


def kernel(feature, W_emb, b_emb, Ws, bs, gammas, betas, W_out, b_out, edge_index):
    pad = jnp.full((2, EPAD - E), N, _i32)
    eflat = jnp.concatenate([edge_index.astype(_i32), pad], axis=1)
    src = eflat[0].reshape(NW, NGRP, G, CH)
    dst = eflat[1].reshape(NW, NGRP, G, CH)
    eidx = jnp.stack([src, dst], axis=2)

    feat_p = jnp.zeros((NP, D), _f32).at[:N].set(feature)

    dego, degi = _deg_kernel(eidx)
    ns, nd = _norm_call(dego.reshape(NC, NP, 1), degi.reshape(NC, NP, 1))

    h, hn = _embed_call(feat_p, W_emb, b_emb.reshape(1, H), ns)
    acc = jnp.broadcast_to(b_out.reshape(1, C), (NP, C))

    for l in range(L):
        p = _agg_kernel(hn, eidx)
        y, s = _layer_a_call(p, nd, Ws[l], bs[l].reshape(1, H))
        h, hn, acc = _layer_b_call(
            y, s, gammas[l].reshape(1, H), betas[l].reshape(1, H),
            h, ns, W_out[l * H:(l + 1) * H], acc)

    return acc[:N]


# speedup vs baseline: 2.7481x; 2.7481x over previous
import functools

import jax
import jax.numpy as jnp
from jax import lax
from jax.experimental import pallas as pl
from jax.experimental.pallas import tpu as pltpu
from jax.experimental.pallas import tpu_sc as plsc

N = 10000
E = 320000
D = 128
H = 128
L = 4
C = 16

NP = 10240
NC = 2
NS = 16
NW = NC * NS
EW = 10240
EPAD = NW * EW
CH = 40
KCH = EW // CH
G = 32
NGRP = KCH // G
NBUF = 4
NT = NP // NS
RB = 1024
NRB = NP // RB

_f32 = jnp.float32
_i32 = jnp.int32

_sc_mesh = plsc.VectorSubcoreMesh(core_axis_name="c", subcore_axis_name="s")


def _fill(ref, rows, value):
    v = jnp.full((16,), value, dtype=_f32)
    if ref.shape == (rows,):
        offs = list(range(0, (rows // 16) * 16, 16))
        if rows % 16:
            offs.append(rows - 16)
        for o in offs:
            ref[pl.ds(o, 16)] = v
    else:
        cols = ref.shape[1]
        for i in range(rows):
            for k in range(cols // 16):
                ref[i, pl.ds(k * 16, 16)] = v


@functools.partial(
    pl.kernel,
    mesh=_sc_mesh,
    out_type=(
        jax.ShapeDtypeStruct((NC * NP,), _f32),
        jax.ShapeDtypeStruct((NC * NP,), _f32),
    ),
    scratch_types=[
        pltpu.VMEM((NGRP, 2, G, CH), _i32),
        pltpu.VMEM((CH,), _f32),
        pltpu.VMEM((NT,), _f32),
        pltpu.VMEM_SHARED((NP,), _f32),
        pltpu.VMEM_SHARED((NP,), _f32),
    ],
)
def _deg_kernel(eidx_hbm, dego_hbm, degi_hbm,
                eidx, ones_v, zrow, sh_o, sh_i):
    cid = lax.axis_index("c")
    sid = lax.axis_index("s")
    wid = sid * NC + cid
    tbase = pl.multiple_of(sid * NT, NT)
    obase = pl.multiple_of(cid * NP + sid * NT, NT)

    pltpu.sync_copy(eidx_hbm.at[wid], eidx)
    _fill(ones_v, CH, 1.0)
    _fill(zrow, NT, 0.0)
    pltpu.sync_copy(zrow, sh_o.at[pl.ds(tbase, NT)])
    pltpu.sync_copy(zrow, sh_i.at[pl.ds(tbase, NT)])
    plsc.subcore_barrier()

    def body(g, _):
        def inner(j, _2):
            pltpu.sync_copy(ones_v, sh_o.at[eidx.at[g, 0, j]], add=True)
            pltpu.sync_copy(ones_v, sh_i.at[eidx.at[g, 1, j]], add=True)
            return _2

        return lax.fori_loop(0, G, inner, _)

    lax.fori_loop(0, NGRP, body, None)
    plsc.subcore_barrier()
    pltpu.sync_copy(sh_o.at[pl.ds(tbase, NT)], dego_hbm.at[pl.ds(obase, NT)])
    pltpu.sync_copy(sh_i.at[pl.ds(tbase, NT)], degi_hbm.at[pl.ds(obase, NT)])


@functools.partial(
    pl.kernel,
    mesh=_sc_mesh,
    out_type=jax.ShapeDtypeStruct((NC, NP, H), _f32),
    scratch_types=[
        pltpu.VMEM((2, G, CH), _i32),
    ] + [pltpu.VMEM((CH, H), _f32) for _ in range(NBUF)]
    + [pltpu.VMEM_SHARED((NP, H), _f32)]
    + [pltpu.SemaphoreType.DMA for _ in range(NBUF)],
)
def _agg_kernel(hn_hbm, eidx_hbm, out_hbm,
                idx2, r0, r1, r2, r3, sh_agg,
                g0, g1, g2, g3):
    cid = lax.axis_index("c")
    sid = lax.axis_index("s")
    wid = sid * NC + cid
    tbase = pl.multiple_of(sid * NT, NT)
    rbufs = (r0, r1, r2, r3)
    gsems = (g0, g1, g2, g3)

    _fill(r0, CH, 0.0)
    for k in range(NT // CH):
        pltpu.sync_copy(r0, sh_agg.at[pl.ds(tbase + k * CH, CH)])
    plsc.subcore_barrier()

    def group(g, _):
        pltpu.sync_copy(eidx_hbm.at[wid, g], idx2)
        for x in range(NBUF - 1):
            pltpu.async_copy(hn_hbm.at[idx2.at[0, x]], rbufs[x], gsems[x])
        for c in range(G):
            b = c % NBUF
            b2 = (c - 1) % NBUF
            pltpu.make_async_copy(hn_hbm.at[idx2.at[0, c]], rbufs[b], gsems[b]).wait()
            if c + NBUF - 1 < G:
                pltpu.async_copy(hn_hbm.at[idx2.at[0, c + NBUF - 1]],
                                 rbufs[b2], gsems[b2])
            pltpu.sync_copy(rbufs[b], sh_agg.at[idx2.at[1, c]], add=True)
        return _

    lax.fori_loop(0, NGRP, group, None)
    plsc.subcore_barrier()
    for k in range(NT // CH):
        sl = pl.ds(tbase + k * CH, CH)
        pltpu.sync_copy(sh_agg.at[sl], out_hbm.at[cid, sl])


def _norm_body(dego_ref, degi_ref, ns_ref, nd_ref):
    do = dego_ref[0] + dego_ref[1]
    di = degi_ref[0] + degi_ref[1]
    ns_ref[...] = jnp.where(do > 0, lax.rsqrt(do), 0.0)
    nd_ref[...] = jnp.where(di > 0, lax.rsqrt(di), 0.0)


def _norm_call(dego, degi):
    return pl.pallas_call(
        _norm_body,
        grid=(NRB,),
        in_specs=[
            pl.BlockSpec((NC, RB, 1), lambda i: (0, i, 0)),
            pl.BlockSpec((NC, RB, 1), lambda i: (0, i, 0)),
        ],
        out_specs=[
            pl.BlockSpec((RB, 1), lambda i: (i, 0)),
            pl.BlockSpec((RB, 1), lambda i: (i, 0)),
        ],
        out_shape=[
            jax.ShapeDtypeStruct((NP, 1), _f32),
            jax.ShapeDtypeStruct((NP, 1), _f32),
        ],
    )(dego, degi)


def _embed_body(x_ref, w_ref, b_ref, ns_ref, h_ref, hn_ref):
    h = jnp.dot(x_ref[...], w_ref[...], preferred_element_type=_f32) + b_ref[...]
    h_ref[...] = h
    hn_ref[...] = h * ns_ref[...]


def _embed_call(x, w, b, ns):
    return pl.pallas_call(
        _embed_body,
        grid=(NRB,),
        in_specs=[
            pl.BlockSpec((RB, D), lambda i: (i, 0)),
            pl.BlockSpec((D, H), lambda i: (0, 0)),
            pl.BlockSpec((1, H), lambda i: (0, 0)),
            pl.BlockSpec((RB, 1), lambda i: (i, 0)),
        ],
        out_specs=[
            pl.BlockSpec((RB, H), lambda i: (i, 0)),
            pl.BlockSpec((RB, H), lambda i: (i, 0)),
        ],
        out_shape=[
            jax.ShapeDtypeStruct((NP, H), _f32),
            jax.ShapeDtypeStruct((NP, H), _f32),
        ],
    )(x, w, b, ns)


def _layer_a_body(p_ref, nd_ref, w_ref, b_ref, y_ref, s_ref):
    i = pl.program_id(0)
    aggn = (p_ref[0] + p_ref[1]) * nd_ref[...]
    y = jnp.dot(aggn, w_ref[...], preferred_element_type=_f32) + b_ref[...]
    y_ref[...] = y
    rows = i * RB + lax.broadcasted_iota(_i32, (RB, 1), 0)
    ym = jnp.where(rows < N, y, 0.0)
    sb = jnp.concatenate(
        [jnp.sum(ym, axis=0, keepdims=True),
         jnp.sum(ym * ym, axis=0, keepdims=True)], axis=0)

    @pl.when(i == 0)
    def _():
        s_ref[...] = sb

    @pl.when(i > 0)
    def _():
        s_ref[...] += sb


def _layer_a_call(p, nd, w, b):
    return pl.pallas_call(
        _layer_a_body,
        grid=(NRB,),
        in_specs=[
            pl.BlockSpec((NC, RB, H), lambda i: (0, i, 0)),
            pl.BlockSpec((RB, 1), lambda i: (i, 0)),
            pl.BlockSpec((H, H), lambda i: (0, 0)),
            pl.BlockSpec((1, H), lambda i: (0, 0)),
        ],
        out_specs=[
            pl.BlockSpec((RB, H), lambda i: (i, 0)),
            pl.BlockSpec((2, H), lambda i: (0, 0)),
        ],
        out_shape=[
            jax.ShapeDtypeStruct((NP, H), _f32),
            jax.ShapeDtypeStruct((2, H), _f32),
        ],
    )(p, nd, w, b)


def _layer_b_body(y_ref, s_ref, g_ref, be_ref, hin_ref, ns_ref, wo_ref, acc_ref,
                  h_ref, hn_ref, accout_ref):
    s = s_ref[...]
    mean = s[0:1] * (1.0 / N)
    var = s[1:2] * (1.0 / N) - mean * mean
    rstd = lax.rsqrt(var + 1e-5)
    y = (y_ref[...] - mean) * rstd * g_ref[...] + be_ref[...]
    hb = jnp.maximum(y, 0.0) + hin_ref[...]
    h_ref[...] = hb
    hn_ref[...] = hb * ns_ref[...]
    accout_ref[...] = acc_ref[...] + jnp.dot(hb, wo_ref[...], preferred_element_type=_f32)


def _layer_b_call(y, s, g, be, hin, ns, wo, acc):
    return pl.pallas_call(
        _layer_b_body,
        grid=(NRB,),
        in_specs=[
            pl.BlockSpec((RB, H), lambda i: (i, 0)),
            pl.BlockSpec((2, H), lambda i: (0, 0)),
            pl.BlockSpec((1, H), lambda i: (0, 0)),
            pl.BlockSpec((1, H), lambda i: (0, 0)),
            pl.BlockSpec((RB, H), lambda i: (i, 0)),
            pl.BlockSpec((RB, 1), lambda i: (i, 0)),
            pl.BlockSpec((H, C), lambda i: (0, 0)),
            pl.BlockSpec((RB, C), lambda i: (i, 0)),
        ],
        out_specs=[
            pl.BlockSpec((RB, H), lambda i: (i, 0)),
            pl.BlockSpec((RB, H), lambda i: (i, 0)),
            pl.BlockSpec((RB, C), lambda i: (i, 0)),
        ],
        out_shape=[
            jax.ShapeDtypeStruct((NP, H), _f32),
            jax.ShapeDtypeStruct((NP, H), _f32),
            jax.ShapeDtypeStruct((NP, C), _f32),
        ],
    )(y, s, g, be, hin, ns, wo, acc)


def kernel(feature, W_emb, b_emb, Ws, bs, gammas, betas, W_out, b_out, edge_index):
    pad_r = N + jnp.arange(EPAD - E, dtype=_i32) % (NP - N)
    eflat = jnp.concatenate([edge_index.astype(_i32), jnp.stack([pad_r, pad_r])], axis=1)
    src = eflat[0].reshape(NW, NGRP, G, CH)
    dst = eflat[1].reshape(NW, NGRP, G, CH)
    eidx = jnp.stack([src, dst], axis=2)

    feat_p = jnp.zeros((NP, D), _f32).at[:N].set(feature)

    dego, degi = _deg_kernel(eidx)
    ns, nd = _norm_call(dego.reshape(NC, NP, 1), degi.reshape(NC, NP, 1))

    h, hn = _embed_call(feat_p, W_emb, b_emb.reshape(1, H), ns)
    acc = jnp.broadcast_to(b_out.reshape(1, C), (NP, C))

    for l in range(L):
        p = _agg_kernel(hn, eidx)
        y, s = _layer_a_call(p, nd, Ws[l], bs[l].reshape(1, H))
        h, hn, acc = _layer_b_call(
            y, s, gammas[l].reshape(1, H), betas[l].reshape(1, H),
            h, ns, W_out[l * H:(l + 1) * H], acc)

    return acc[:N]
